# baseline (speedup 1.0000x reference)
import jax
import jax.numpy as jnp
from jax import lax
from jax.experimental import pallas as pl
from jax.experimental.pallas import tpu as pltpu
from jax.experimental.pallas import tpu_sc as plsc

HID = 768
FFN_D = 3072
NE = 8
TOPK = 2
SHARED_SCALE = 0.1
NTOK = 2048
NSLOT = NTOK * TOPK
BLK = 128
NBE = (NSLOT + NE * (BLK - 1)) // BLK + 1
S_EXP = NBE * BLK
NBS = NTOK // BLK
NBT = NBE + NBS
SG = S_EXP + NTOK

NCC = 2
NSC = 16
NW = NCC * NSC
RPW = SG // NW
GCHUNK = 112
TPW = NTOK // NW
CCH = 32


def _cumsum_rows(a):
    n = a.shape[0]
    s = 1
    while s < n:
        a = a + jnp.concatenate(
            [jnp.zeros((s, a.shape[1]), a.dtype), a[:-s, :]], axis=0)
        s *= 2
    return a


def _lane_excl_cumsum(a):
    r = a
    s = 1
    while s < a.shape[1]:
        r = r + jnp.concatenate(
            [jnp.zeros((1, s), a.dtype), r[:, :-s]], axis=1)
        s *= 2
    return r - a


def _router_body(x_ref, wg_ref, pos_ref, g_ref, me_ref, mv_ref):
    x = x_ref[...]
    logits = jnp.dot(x, wg_ref[...], preferred_element_type=jnp.float32)
    probs = jax.nn.softmax(logits, axis=-1)
    iota = lax.broadcasted_iota(jnp.int32, probs.shape, 1)
    m1 = jnp.max(probs, axis=1, keepdims=True)
    i1 = jnp.min(jnp.where(probs == m1, iota, NE), axis=1, keepdims=True)
    oh1 = iota == i1
    pm = jnp.where(oh1, -1.0, probs)
    m2 = jnp.max(pm, axis=1, keepdims=True)
    i2 = jnp.min(jnp.where(pm == m2, iota, NE), axis=1, keepdims=True)
    oh2 = iota == i2
    denom = m1 + m2 + 1e-9
    f1 = oh1.astype(jnp.float32)
    f2 = oh2.astype(jnp.float32)

    inc1 = _cumsum_rows(f1)
    inc2 = _cumsum_rows(f2)
    excl1 = inc1 - f1
    excl2 = inc2 - f2
    tot1 = inc1[NTOK - 1:NTOK, :]
    tot2 = inc2[NTOK - 1:NTOK, :]
    counts = tot1 + tot2
    nblk = jnp.floor((counts + (BLK - 1)) * (1.0 / BLK))
    blk_off = _lane_excl_cumsum(nblk)
    row_off = blk_off * BLK

    pos_k0 = jnp.sum(f1 * row_off, 1, keepdims=True) + \
        jnp.sum(f1 * excl1, 1, keepdims=True)
    pos_k1 = jnp.sum(f2 * row_off, 1, keepdims=True) + \
        jnp.sum(f2 * (tot1 + excl2), 1, keepdims=True)
    pos_ref[...] = jnp.concatenate([pos_k0, pos_k1], axis=0).astype(jnp.int32)
    g_ref[...] = jnp.concatenate([m1 / denom, m2 / denom], axis=0)

    nbu = jnp.sum(nblk, axis=1, keepdims=True)
    bio = lax.broadcasted_iota(jnp.int32, (NBE, NE), 0).astype(jnp.float32)
    me_exp = (jnp.sum((bio >= blk_off).astype(jnp.float32), 1, keepdims=True)
              - 1.0).astype(jnp.int32)
    mv_exp = (lax.broadcasted_iota(jnp.int32, (NBE, 1), 0).astype(jnp.float32)
              < nbu).astype(jnp.int32)
    me_ref[...] = jnp.concatenate(
        [me_exp, jnp.full((NBS, 1), NE, jnp.int32)], axis=0)
    mv_ref[...] = jnp.concatenate(
        [mv_exp, jnp.ones((NBS, 1), jnp.int32)], axis=0)


def _router(x, Wg):
    return pl.pallas_call(
        _router_body,
        out_shape=[
            jax.ShapeDtypeStruct((NSLOT, 1), jnp.int32),
            jax.ShapeDtypeStruct((NSLOT, 1), jnp.float32),
            jax.ShapeDtypeStruct((NBT, 1), jnp.int32),
            jax.ShapeDtypeStruct((NBT, 1), jnp.int32),
        ],
    )(x, Wg)


def _dispatch_body(p_hbm, g_hbm, x_hbm, xg_hbm, gg_hbm,
                   p_loc, g_loc, src_w, g_w, rows_v, sem):
    c = lax.axis_index("c")
    s = lax.axis_index("s")
    wid = s * NCC + c
    base = wid * RPW
    pltpu.sync_copy(p_hbm, p_loc)
    pltpu.sync_copy(g_hbm, g_loc)
    lanes = lax.iota(jnp.int32, 16)

    def init_j(j, carry):
        rv = base + j * 16 + lanes
        in_tail = rv >= S_EXP
        src_w[pl.ds(j * 16, 16)] = jnp.where(in_tail, rv - S_EXP, 0)
        g_w[pl.ds(j * 16, 16)] = jnp.where(
            in_tail, jnp.float32(SHARED_SCALE), jnp.float32(0.0))
        return carry
    lax.fori_loop(0, RPW // 16, init_j, 0)

    def scat_j(j, carry):
        s0 = j * 16
        pv = p_loc[pl.ds(s0, 16)]
        gv = g_loc[pl.ds(s0, 16)]
        sv = s0 + lanes
        tok = jnp.where(sv >= NTOK, sv - NTOK, sv)
        loc = pv - base
        m = (loc >= 0) & (loc < RPW)
        locc = jnp.clip(loc, 0, RPW - 1)
        plsc.store_scatter(src_w, [locc], tok, mask=m)
        plsc.store_scatter(g_w, [locc], gv, mask=m)
        return carry
    lax.fori_loop(0, NSLOT // 16, scat_j, 0)

    pltpu.sync_copy(g_w, gg_hbm.at[pl.ds(base, RPW)])
    for ci in range(RPW // GCHUNK):
        idx = src_w.at[pl.ds(ci * GCHUNK, GCHUNK)]
        pltpu.async_copy(x_hbm.at[idx], rows_v, sem).wait()
        pltpu.sync_copy(rows_v, xg_hbm.at[pl.ds(base + ci * GCHUNK, GCHUNK)])


def _dispatch(p1d, g1d, x):
    mesh = plsc.VectorSubcoreMesh(core_axis_name="c", subcore_axis_name="s")
    return pl.kernel(
        _dispatch_body,
        out_type=[
            jax.ShapeDtypeStruct((SG, HID), jnp.float32),
            jax.ShapeDtypeStruct((SG,), jnp.float32),
        ],
        mesh=mesh,
        scratch_types=[
            pltpu.VMEM((NSLOT,), jnp.int32),
            pltpu.VMEM((NSLOT,), jnp.float32),
            pltpu.VMEM((RPW,), jnp.int32),
            pltpu.VMEM((RPW,), jnp.float32),
            pltpu.VMEM((GCHUNK, HID), jnp.float32),
            pltpu.SemaphoreType.DMA,
        ],
        compiler_params=pltpu.CompilerParams(needs_layout_passes=False),
    )(p1d, g1d, x)


def _ffn_body(me_ref, mv_ref, xg_ref, g_ref, w1_ref, b1_ref, w2_ref, b2_ref,
              yg_ref):
    b = pl.program_id(0)

    @pl.when(mv_ref[b] > 0)
    def _():
        xb = xg_ref[...]
        h = jnp.dot(xb, w1_ref[0], preferred_element_type=jnp.float32) \
            + b1_ref[0]
        h = h * jax.nn.sigmoid(h)
        y = jnp.dot(h, w2_ref[0], preferred_element_type=jnp.float32) \
            + b2_ref[0]
        yg_ref[...] = jnp.reshape(g_ref[0, 0], (BLK, 1)) * y


def _ffn(me, mv, xg, ggrp3, W1c, b1c, W2c, b2c):
    grid_spec = pltpu.PrefetchScalarGridSpec(
        num_scalar_prefetch=2,
        grid=(NBT,),
        in_specs=[
            pl.BlockSpec((BLK, HID), lambda b, me, mv: (b, 0)),
            pl.BlockSpec((1, 1, BLK), lambda b, me, mv: (b, 0, 0)),
            pl.BlockSpec((1, HID, FFN_D), lambda b, me, mv: (me[b], 0, 0)),
            pl.BlockSpec((1, 1, FFN_D), lambda b, me, mv: (me[b], 0, 0)),
            pl.BlockSpec((1, FFN_D, HID), lambda b, me, mv: (me[b], 0, 0)),
            pl.BlockSpec((1, 1, HID), lambda b, me, mv: (me[b], 0, 0)),
        ],
        out_specs=pl.BlockSpec((BLK, HID), lambda b, me, mv: (b, 0)),
    )
    return pl.pallas_call(
        _ffn_body,
        grid_spec=grid_spec,
        out_shape=jax.ShapeDtypeStruct((SG, HID), jnp.float32),
    )(me, mv, xg, ggrp3, W1c, b1c, W2c, b2c)


def _combine_body(p_hbm, yg_hbm, out_hbm, idx0_v, idx1_v, r0, r1, r2,
                  sem0, sem1):
    c = lax.axis_index("c")
    s = lax.axis_index("s")
    wid = s * NCC + c
    t0 = wid * TPW
    pltpu.sync_copy(p_hbm.at[pl.ds(t0, TPW)], idx0_v)
    pltpu.sync_copy(p_hbm.at[pl.ds(NTOK + t0, TPW)], idx1_v)
    for ci in range(TPW // CCH):
        a0 = pltpu.async_copy(
            yg_hbm.at[idx0_v.at[pl.ds(ci * CCH, CCH)]], r0, sem0)
        a1 = pltpu.async_copy(
            yg_hbm.at[idx1_v.at[pl.ds(ci * CCH, CCH)]], r1, sem1)
        pltpu.sync_copy(
            yg_hbm.at[pl.ds(S_EXP + t0 + ci * CCH, CCH)], r2)
        a0.wait()
        a1.wait()

        def addrow(r, carry):
            for l in range(HID // 16):
                sl = pl.ds(l * 16, 16)
                r0[r, sl] = r0[r, sl] + r1[r, sl] + r2[r, sl]
            return carry
        lax.fori_loop(0, CCH, addrow, 0)
        pltpu.sync_copy(r0, out_hbm.at[pl.ds(t0 + ci * CCH, CCH)])


def _combine(p1d, yg):
    mesh = plsc.VectorSubcoreMesh(core_axis_name="c", subcore_axis_name="s")
    return pl.kernel(
        _combine_body,
        out_type=jax.ShapeDtypeStruct((NTOK, HID), jnp.float32),
        mesh=mesh,
        scratch_types=[
            pltpu.VMEM((TPW,), jnp.int32),
            pltpu.VMEM((TPW,), jnp.int32),
            pltpu.VMEM((CCH, HID), jnp.float32),
            pltpu.VMEM((CCH, HID), jnp.float32),
            pltpu.VMEM((CCH, HID), jnp.float32),
            pltpu.SemaphoreType.DMA,
            pltpu.SemaphoreType.DMA,
        ],
        compiler_params=pltpu.CompilerParams(needs_layout_passes=False),
    )(p1d, yg)


def kernel(hidden_states, Wg, W1, b1, W2, b2, Ws1, bs1, Ws2, bs2):
    orig = hidden_states.shape
    x = hidden_states.reshape(-1, orig[-1])
    W1c = jnp.concatenate([W1, Ws1[None]], axis=0)
    W2c = jnp.concatenate([W2, Ws2[None]], axis=0)
    b1c = jnp.concatenate([b1, bs1[None]], axis=0)[:, None, :]
    b2c = jnp.concatenate([b2, bs2[None]], axis=0)[:, None, :]

    pos, g, me, mv = _router(x, Wg)
    p1d = pos.reshape(NSLOT)
    g1d = g.reshape(NSLOT)
    xg, ggrp = _dispatch(p1d, g1d, x)
    ggrp3 = ggrp.reshape(NBT, 1, BLK)
    yg = _ffn(me.reshape(NBT), mv.reshape(NBT), xg, ggrp3, W1c, b1c, W2c, b2c)
    out = _combine(p1d, yg)
    return out.reshape(orig)


# speedup vs baseline: 1.5307x; 1.5307x over previous
import jax
import jax.numpy as jnp
from jax import lax
from jax.experimental import pallas as pl
from jax.experimental.pallas import tpu as pltpu
from jax.experimental.pallas import tpu_sc as plsc

HID = 768
FFN_D = 3072
NE = 8
TOPK = 2
SHARED_SCALE = 0.1
NTOK = 2048
NSLOT = NTOK * TOPK
BLK = 128
NBE = (NSLOT + NE * (BLK - 1)) // BLK + 1
S_EXP = NBE * BLK
CH = 128
NCH = NTOK // CH

NCC = 2
NSC = 16
NW = NCC * NSC
RPW = S_EXP // NW
GCH = 40
NGC = RPW // GCH
TPW = NTOK // NW
CCH = 32
SBLK = 256


def _router_body(x_ref, wg_ref, pos_ref, g_ref, me_ref, mv_ref):
    x = x_ref[...]
    logits = jnp.dot(x, wg_ref[...], preferred_element_type=jnp.float32)
    probs = jax.nn.softmax(logits, axis=-1)
    iota = lax.broadcasted_iota(jnp.int32, probs.shape, 1)
    m1 = jnp.max(probs, axis=1, keepdims=True)
    i1 = jnp.min(jnp.where(probs == m1, iota, NE), axis=1, keepdims=True)
    oh1 = iota == i1
    pm = jnp.where(oh1, -1.0, probs)
    m2 = jnp.max(pm, axis=1, keepdims=True)
    i2 = jnp.min(jnp.where(pm == m2, iota, NE), axis=1, keepdims=True)
    oh2 = iota == i2
    denom = m1 + m2 + 1e-9
    f1 = oh1.astype(jnp.float32)
    f2 = oh2.astype(jnp.float32)

    a = jnp.concatenate([f1, f2], axis=1)
    tl = (lax.broadcasted_iota(jnp.int32, (CH, CH), 0)
          >= lax.broadcasted_iota(jnp.int32, (CH, CH), 1)).astype(jnp.float32)
    stl = (lax.broadcasted_iota(jnp.int32, (NCH, NCH), 0)
           > lax.broadcasted_iota(jnp.int32, (NCH, NCH), 1)).astype(jnp.float32)
    incs = [jnp.dot(tl, a[j * CH:(j + 1) * CH, :],
                    preferred_element_type=jnp.float32) for j in range(NCH)]
    tots = jnp.concatenate([ic[CH - 1:CH, :] for ic in incs], axis=0)
    offs = jnp.dot(stl, tots, preferred_element_type=jnp.float32)
    inc = jnp.concatenate(
        [incs[j] + offs[j:j + 1, :] for j in range(NCH)], axis=0)

    inc1 = inc[:, :NE]
    inc2 = inc[:, NE:]
    excl1 = inc1 - f1
    excl2 = inc2 - f2
    tot1 = inc1[NTOK - 1:NTOK, :]
    tot2 = inc2[NTOK - 1:NTOK, :]
    counts = tot1 + tot2
    nblk = jnp.floor((counts + (BLK - 1)) * (1.0 / BLK))
    s = 1
    r = nblk
    while s < NE:
        r = r + jnp.concatenate(
            [jnp.zeros((1, s), r.dtype), r[:, :-s]], axis=1)
        s *= 2
    blk_off = r - nblk
    row_off = blk_off * BLK

    pos_k0 = jnp.sum(f1 * row_off, 1, keepdims=True) + \
        jnp.sum(f1 * excl1, 1, keepdims=True)
    pos_k1 = jnp.sum(f2 * row_off, 1, keepdims=True) + \
        jnp.sum(f2 * (tot1 + excl2), 1, keepdims=True)
    pos_ref[...] = jnp.concatenate([pos_k0, pos_k1], axis=0).astype(jnp.int32)
    g_ref[...] = jnp.concatenate([m1 / denom, m2 / denom], axis=0)

    nbu = jnp.sum(nblk, axis=1, keepdims=True)
    bio = lax.broadcasted_iota(jnp.int32, (NBE, NE), 0).astype(jnp.float32)
    me_ref[...] = (jnp.sum((bio >= blk_off).astype(jnp.float32), 1,
                           keepdims=True) - 1.0).astype(jnp.int32)
    mv_ref[...] = (lax.broadcasted_iota(jnp.int32, (NBE, 1), 0)
                   .astype(jnp.float32) < nbu).astype(jnp.int32)


def _router(x, Wg):
    return pl.pallas_call(
        _router_body,
        out_shape=[
            jax.ShapeDtypeStruct((NSLOT, 1), jnp.int32),
            jax.ShapeDtypeStruct((NSLOT, 1), jnp.float32),
            jax.ShapeDtypeStruct((NBE, 1), jnp.int32),
            jax.ShapeDtypeStruct((NBE, 1), jnp.int32),
        ],
    )(x, Wg)


def _dispatch_body(p_hbm, g_hbm, x_hbm, xg_hbm, gg_hbm,
                   p_loc, g_loc, src_w, g_w, buf0, buf1,
                   gs0, gs1, ws0, ws1):
    c = lax.axis_index("c")
    s = lax.axis_index("s")
    wid = s * NCC + c
    base = wid * RPW
    pltpu.sync_copy(p_hbm, p_loc)
    pltpu.sync_copy(g_hbm, g_loc)
    lanes = lax.iota(jnp.int32, 16)

    def init_j(j, carry):
        src_w[pl.ds(j * 16, 16)] = jnp.zeros((16,), jnp.int32)
        g_w[pl.ds(j * 16, 16)] = jnp.zeros((16,), jnp.float32)
        return carry
    lax.fori_loop(0, RPW // 16, init_j, 0)

    def scat_j(j, carry):
        s0 = j * 16
        pv = p_loc[pl.ds(s0, 16)]
        gv = g_loc[pl.ds(s0, 16)]
        sv = s0 + lanes
        tok = jnp.where(sv >= NTOK, sv - NTOK, sv)
        loc = pv - base
        m = (loc >= 0) & (loc < RPW)
        locc = jnp.clip(loc, 0, RPW - 1)
        plsc.store_scatter(src_w, [locc], tok, mask=m)
        plsc.store_scatter(g_w, [locc], gv, mask=m)
        return carry
    lax.fori_loop(0, NSLOT // 16, scat_j, 0)

    pltpu.sync_copy(g_w, gg_hbm.at[pl.ds(base, RPW)])

    bufs = [buf0, buf1]
    gsems = [gs0, gs1]
    wsems = [ws0, ws1]
    copies = {}
    wcopies = {}
    copies[0] = pltpu.async_copy(
        x_hbm.at[src_w.at[pl.ds(0, GCH)]], bufs[0], gsems[0])
    for ci in range(NGC):
        if ci + 1 < NGC:
            if ci >= 1:
                wcopies[ci - 1].wait()
            nb = (ci + 1) % 2
            copies[ci + 1] = pltpu.async_copy(
                x_hbm.at[src_w.at[pl.ds((ci + 1) * GCH, GCH)]],
                bufs[nb], gsems[nb])
        copies[ci].wait()
        wcopies[ci] = pltpu.async_copy(
            bufs[ci % 2], xg_hbm.at[pl.ds(base + ci * GCH, GCH)],
            wsems[ci % 2])
    wcopies[NGC - 2].wait()
    wcopies[NGC - 1].wait()


def _dispatch(p1d, g1d, x):
    mesh = plsc.VectorSubcoreMesh(core_axis_name="c", subcore_axis_name="s")
    return pl.kernel(
        _dispatch_body,
        out_type=[
            jax.ShapeDtypeStruct((S_EXP, HID), jnp.float32),
            jax.ShapeDtypeStruct((S_EXP,), jnp.float32),
        ],
        mesh=mesh,
        scratch_types=[
            pltpu.VMEM((NSLOT,), jnp.int32),
            pltpu.VMEM((NSLOT,), jnp.float32),
            pltpu.VMEM((RPW,), jnp.int32),
            pltpu.VMEM((RPW,), jnp.float32),
            pltpu.VMEM((GCH, HID), jnp.float32),
            pltpu.VMEM((GCH, HID), jnp.float32),
            pltpu.SemaphoreType.DMA,
            pltpu.SemaphoreType.DMA,
            pltpu.SemaphoreType.DMA,
            pltpu.SemaphoreType.DMA,
        ],
        compiler_params=pltpu.CompilerParams(needs_layout_passes=False),
    )(p1d, g1d, x)


def _shared_body(x_ref, w1_ref, b1_ref, w2_ref, b2_ref, ys_ref):
    xb = x_ref[...]
    h = jnp.dot(xb, w1_ref[...], preferred_element_type=jnp.float32) \
        + b1_ref[...]
    h = h * jax.nn.sigmoid(h)
    y = jnp.dot(h, w2_ref[...], preferred_element_type=jnp.float32) \
        + b2_ref[...]
    ys_ref[...] = SHARED_SCALE * y


def _shared(x, Ws1, bs1, Ws2, bs2):
    return pl.pallas_call(
        _shared_body,
        grid=(NTOK // SBLK,),
        in_specs=[
            pl.BlockSpec((SBLK, HID), lambda t: (t, 0)),
            pl.BlockSpec((HID, FFN_D), lambda t: (0, 0)),
            pl.BlockSpec((1, FFN_D), lambda t: (0, 0)),
            pl.BlockSpec((FFN_D, HID), lambda t: (0, 0)),
            pl.BlockSpec((1, HID), lambda t: (0, 0)),
        ],
        out_specs=pl.BlockSpec((SBLK, HID), lambda t: (t, 0)),
        out_shape=jax.ShapeDtypeStruct((NTOK, HID), jnp.float32),
    )(x, Ws1, bs1, Ws2, bs2)


def _ffn_body(me_ref, mv_ref, xg_ref, g_ref, w1_ref, b1_ref, w2_ref, b2_ref,
              yg_ref):
    b = pl.program_id(0)

    @pl.when(mv_ref[b] > 0)
    def _():
        xb = xg_ref[...]
        h = jnp.dot(xb, w1_ref[0], preferred_element_type=jnp.float32) \
            + b1_ref[0]
        h = h * jax.nn.sigmoid(h)
        y = jnp.dot(h, w2_ref[0], preferred_element_type=jnp.float32) \
            + b2_ref[0]
        yg_ref[...] = jnp.reshape(g_ref[0, 0], (BLK, 1)) * y


def _ffn(me, mv, xg, ggrp3, W1, b13, W2, b23):
    grid_spec = pltpu.PrefetchScalarGridSpec(
        num_scalar_prefetch=2,
        grid=(NBE,),
        in_specs=[
            pl.BlockSpec((BLK, HID), lambda b, me, mv: (b, 0)),
            pl.BlockSpec((1, 1, BLK), lambda b, me, mv: (b, 0, 0)),
            pl.BlockSpec((1, HID, FFN_D), lambda b, me, mv: (me[b], 0, 0)),
            pl.BlockSpec((1, 1, FFN_D), lambda b, me, mv: (me[b], 0, 0)),
            pl.BlockSpec((1, FFN_D, HID), lambda b, me, mv: (me[b], 0, 0)),
            pl.BlockSpec((1, 1, HID), lambda b, me, mv: (me[b], 0, 0)),
        ],
        out_specs=pl.BlockSpec((BLK, HID), lambda b, me, mv: (b, 0)),
    )
    return pl.pallas_call(
        _ffn_body,
        grid_spec=grid_spec,
        out_shape=jax.ShapeDtypeStruct((S_EXP, HID), jnp.float32),
    )(me, mv, xg, ggrp3, W1, b13, W2, b23)


def _combine_body(p_hbm, yg_hbm, ys_hbm, out_hbm, idx0_v, idx1_v, r0, r1, r2,
                  sem0, sem1):
    c = lax.axis_index("c")
    s = lax.axis_index("s")
    wid = s * NCC + c
    t0 = wid * TPW
    pltpu.sync_copy(p_hbm.at[pl.ds(t0, TPW)], idx0_v)
    pltpu.sync_copy(p_hbm.at[pl.ds(NTOK + t0, TPW)], idx1_v)
    for ci in range(TPW // CCH):
        a0 = pltpu.async_copy(
            yg_hbm.at[idx0_v.at[pl.ds(ci * CCH, CCH)]], r0, sem0)
        a1 = pltpu.async_copy(
            yg_hbm.at[idx1_v.at[pl.ds(ci * CCH, CCH)]], r1, sem1)
        pltpu.sync_copy(ys_hbm.at[pl.ds(t0 + ci * CCH, CCH)], r2)
        a0.wait()
        a1.wait()

        def addrow(rr, carry):
            for l in range(HID // 16):
                sl = pl.ds(l * 16, 16)
                r0[rr, sl] = r0[rr, sl] + r1[rr, sl] + r2[rr, sl]
            return carry
        lax.fori_loop(0, CCH, addrow, 0)
        pltpu.sync_copy(r0, out_hbm.at[pl.ds(t0 + ci * CCH, CCH)])


def _combine(p1d, yg, ys):
    mesh = plsc.VectorSubcoreMesh(core_axis_name="c", subcore_axis_name="s")
    return pl.kernel(
        _combine_body,
        out_type=jax.ShapeDtypeStruct((NTOK, HID), jnp.float32),
        mesh=mesh,
        scratch_types=[
            pltpu.VMEM((TPW,), jnp.int32),
            pltpu.VMEM((TPW,), jnp.int32),
            pltpu.VMEM((CCH, HID), jnp.float32),
            pltpu.VMEM((CCH, HID), jnp.float32),
            pltpu.VMEM((CCH, HID), jnp.float32),
            pltpu.SemaphoreType.DMA,
            pltpu.SemaphoreType.DMA,
        ],
        compiler_params=pltpu.CompilerParams(needs_layout_passes=False),
    )(p1d, yg, ys)


def kernel(hidden_states, Wg, W1, b1, W2, b2, Ws1, bs1, Ws2, bs2):
    orig = hidden_states.shape
    x = hidden_states.reshape(-1, orig[-1])
    b13 = b1[:, None, :]
    b23 = b2[:, None, :]

    pos, g, me, mv = _router(x, Wg)
    p1d = pos.reshape(NSLOT)
    g1d = g.reshape(NSLOT)
    xg, ggrp = _dispatch(p1d, g1d, x)
    ys = _shared(x, Ws1, bs1[None, :], Ws2, bs2[None, :])
    ggrp3 = ggrp.reshape(NBE, 1, BLK)
    yg = _ffn(me.reshape(NBE), mv.reshape(NBE), xg, ggrp3, W1, b13, W2, b23)
    out = _combine(p1d, yg, ys)
    return out.reshape(orig)


# speedup vs baseline: 1.5319x; 1.0008x over previous
import jax
import jax.numpy as jnp
from jax import lax
from jax.experimental import pallas as pl
from jax.experimental.pallas import tpu as pltpu
from jax.experimental.pallas import tpu_sc as plsc

HID = 768
FFN_D = 3072
NE = 8
TOPK = 2
SHARED_SCALE = 0.1
NTOK = 2048
NSLOT = NTOK * TOPK
BLK = 128
NBE = (NSLOT + NE * (BLK - 1)) // BLK + 1
S_EXP = NBE * BLK
CH = 128
NCH = NTOK // CH

NCC = 2
NSC = 16
NW = NCC * NSC
RPW = S_EXP // NW
GCH = 40
NGC = RPW // GCH
TPW = NTOK // NW
CCH = 32
SBLK = 256


def _router_body(x_ref, wg_ref, pos_ref, g_ref, me_ref, mv_ref):
    x = x_ref[...]
    logits = jnp.dot(x, wg_ref[...], preferred_element_type=jnp.float32)
    probs = jax.nn.softmax(logits, axis=-1)
    iota = lax.broadcasted_iota(jnp.int32, probs.shape, 1)
    m1 = jnp.max(probs, axis=1, keepdims=True)
    i1 = jnp.min(jnp.where(probs == m1, iota, NE), axis=1, keepdims=True)
    oh1 = iota == i1
    pm = jnp.where(oh1, -1.0, probs)
    m2 = jnp.max(pm, axis=1, keepdims=True)
    i2 = jnp.min(jnp.where(pm == m2, iota, NE), axis=1, keepdims=True)
    oh2 = iota == i2
    denom = m1 + m2 + 1e-9
    f1 = oh1.astype(jnp.float32)
    f2 = oh2.astype(jnp.float32)

    a = jnp.concatenate([f1, f2], axis=1)
    tl = (lax.broadcasted_iota(jnp.int32, (CH, CH), 0)
          >= lax.broadcasted_iota(jnp.int32, (CH, CH), 1)).astype(jnp.float32)
    stl = (lax.broadcasted_iota(jnp.int32, (NCH, NCH), 0)
           > lax.broadcasted_iota(jnp.int32, (NCH, NCH), 1)).astype(jnp.float32)
    incs = [jnp.dot(tl, a[j * CH:(j + 1) * CH, :],
                    preferred_element_type=jnp.float32) for j in range(NCH)]
    tots = jnp.concatenate([ic[CH - 1:CH, :] for ic in incs], axis=0)
    offs = jnp.dot(stl, tots, preferred_element_type=jnp.float32)
    inc = jnp.concatenate(
        [incs[j] + offs[j:j + 1, :] for j in range(NCH)], axis=0)

    inc1 = inc[:, :NE]
    inc2 = inc[:, NE:]
    excl1 = inc1 - f1
    excl2 = inc2 - f2
    tot1 = inc1[NTOK - 1:NTOK, :]
    tot2 = inc2[NTOK - 1:NTOK, :]
    counts = tot1 + tot2
    nblk = jnp.floor((counts + (BLK - 1)) * (1.0 / BLK))
    s = 1
    r = nblk
    while s < NE:
        r = r + jnp.concatenate(
            [jnp.zeros((1, s), r.dtype), r[:, :-s]], axis=1)
        s *= 2
    blk_off = r - nblk
    row_off = blk_off * BLK

    pos_k0 = jnp.sum(f1 * row_off, 1, keepdims=True) + \
        jnp.sum(f1 * excl1, 1, keepdims=True)
    pos_k1 = jnp.sum(f2 * row_off, 1, keepdims=True) + \
        jnp.sum(f2 * (tot1 + excl2), 1, keepdims=True)
    pos_ref[...] = jnp.concatenate([pos_k0, pos_k1], axis=0).astype(jnp.int32)
    g_ref[...] = jnp.concatenate([m1 / denom, m2 / denom], axis=0)

    nbu = jnp.sum(nblk, axis=1, keepdims=True)
    bio = lax.broadcasted_iota(jnp.int32, (NBE, NE), 0).astype(jnp.float32)
    me_ref[...] = (jnp.sum((bio >= blk_off).astype(jnp.float32), 1,
                           keepdims=True) - 1.0).astype(jnp.int32)
    mv_ref[...] = (lax.broadcasted_iota(jnp.int32, (NBE, 1), 0)
                   .astype(jnp.float32) < nbu).astype(jnp.int32)


def _router(x, Wg):
    return pl.pallas_call(
        _router_body,
        out_shape=[
            jax.ShapeDtypeStruct((NSLOT, 1), jnp.int32),
            jax.ShapeDtypeStruct((NSLOT, 1), jnp.float32),
            jax.ShapeDtypeStruct((NBE, 1), jnp.int32),
            jax.ShapeDtypeStruct((NBE, 1), jnp.int32),
        ],
    )(x, Wg)


NRING = 3


def _dispatch_body(p_hbm, g_hbm, x_hbm, xg_hbm, gg_hbm,
                   p_loc, g_loc, src_w, g_w, b0, b1, b2, *sems):
    c = lax.axis_index("c")
    s = lax.axis_index("s")
    wid = s * NCC + c
    base = wid * RPW
    pltpu.sync_copy(p_hbm, p_loc)
    pltpu.sync_copy(g_hbm, g_loc)
    lanes = lax.iota(jnp.int32, 16)

    def init_j(j, carry):
        src_w[pl.ds(j * 16, 16)] = jnp.zeros((16,), jnp.int32)
        g_w[pl.ds(j * 16, 16)] = jnp.zeros((16,), jnp.float32)
        return carry
    lax.fori_loop(0, RPW // 16, init_j, 0)

    def scat_j(j, carry):
        s0 = j * 16
        pv = p_loc[pl.ds(s0, 16)]
        gv = g_loc[pl.ds(s0, 16)]
        sv = s0 + lanes
        tok = jnp.where(sv >= NTOK, sv - NTOK, sv)
        loc = pv - base
        m = (loc >= 0) & (loc < RPW)
        locc = jnp.clip(loc, 0, RPW - 1)
        plsc.store_scatter(src_w, [locc], tok, mask=m)
        plsc.store_scatter(g_w, [locc], gv, mask=m)
        return carry
    lax.fori_loop(0, NSLOT // 16, scat_j, 0)

    pltpu.sync_copy(g_w, gg_hbm.at[pl.ds(base, RPW)])

    bufs = [b0, b1, b2]
    gcop = {}
    wcop = {}
    for ci in range(min(NRING, NGC)):
        gcop[ci] = pltpu.async_copy(
            x_hbm.at[src_w.at[pl.ds(ci * GCH, GCH)]], bufs[ci % NRING],
            sems[ci % NRING])
    for ci in range(NGC):
        gcop[ci].wait()
        wcop[ci] = pltpu.async_copy(
            bufs[ci % NRING], xg_hbm.at[pl.ds(base + ci * GCH, GCH)],
            sems[NRING + ci % NRING])
        nx = ci + NRING
        if nx < NGC:
            wcop[ci].wait()
            gcop[nx] = pltpu.async_copy(
                x_hbm.at[src_w.at[pl.ds(nx * GCH, GCH)]], bufs[nx % NRING],
                sems[nx % NRING])
    for ci in range(max(0, NGC - NRING), NGC):
        if ci in wcop:
            wcop[ci].wait()


def _dispatch(p1d, g1d, x):
    mesh = plsc.VectorSubcoreMesh(core_axis_name="c", subcore_axis_name="s")
    return pl.kernel(
        _dispatch_body,
        out_type=[
            jax.ShapeDtypeStruct((S_EXP, HID), jnp.float32),
            jax.ShapeDtypeStruct((S_EXP,), jnp.float32),
        ],
        mesh=mesh,
        scratch_types=[
            pltpu.VMEM((NSLOT,), jnp.int32),
            pltpu.VMEM((NSLOT,), jnp.float32),
            pltpu.VMEM((RPW,), jnp.int32),
            pltpu.VMEM((RPW,), jnp.float32),
            pltpu.VMEM((GCH, HID), jnp.float32),
            pltpu.VMEM((GCH, HID), jnp.float32),
            pltpu.VMEM((GCH, HID), jnp.float32),
        ] + [pltpu.SemaphoreType.DMA] * (2 * NRING),
        compiler_params=pltpu.CompilerParams(needs_layout_passes=False),
    )(p1d, g1d, x)


def _shared_body(x_ref, w1_ref, b1_ref, w2_ref, b2_ref, ys_ref):
    xb = x_ref[...]
    h = jnp.dot(xb, w1_ref[...], preferred_element_type=jnp.float32) \
        + b1_ref[...]
    h = h * jax.nn.sigmoid(h)
    y = jnp.dot(h, w2_ref[...], preferred_element_type=jnp.float32) \
        + b2_ref[...]
    ys_ref[...] = SHARED_SCALE * y


def _shared(x, Ws1, bs1, Ws2, bs2):
    return pl.pallas_call(
        _shared_body,
        grid=(NTOK // SBLK,),
        in_specs=[
            pl.BlockSpec((SBLK, HID), lambda t: (t, 0)),
            pl.BlockSpec((HID, FFN_D), lambda t: (0, 0)),
            pl.BlockSpec((1, FFN_D), lambda t: (0, 0)),
            pl.BlockSpec((FFN_D, HID), lambda t: (0, 0)),
            pl.BlockSpec((1, HID), lambda t: (0, 0)),
        ],
        out_specs=pl.BlockSpec((SBLK, HID), lambda t: (t, 0)),
        out_shape=jax.ShapeDtypeStruct((NTOK, HID), jnp.float32),
    )(x, Ws1, bs1, Ws2, bs2)


def _ffn_body(me_ref, mv_ref, xg_ref, g_ref, w1_ref, b1_ref, w2_ref, b2_ref,
              yg_ref):
    b = pl.program_id(0)

    @pl.when(mv_ref[b] > 0)
    def _():
        xb = xg_ref[...]
        h = jnp.dot(xb, w1_ref[0], preferred_element_type=jnp.float32) \
            + b1_ref[0]
        h = h * jax.nn.sigmoid(h)
        y = jnp.dot(h, w2_ref[0], preferred_element_type=jnp.float32) \
            + b2_ref[0]
        yg_ref[...] = jnp.reshape(g_ref[0, 0], (BLK, 1)) * y


def _ffn(me, mv, xg, ggrp3, W1, b13, W2, b23):
    grid_spec = pltpu.PrefetchScalarGridSpec(
        num_scalar_prefetch=2,
        grid=(NBE,),
        in_specs=[
            pl.BlockSpec((BLK, HID), lambda b, me, mv: (b, 0)),
            pl.BlockSpec((1, 1, BLK), lambda b, me, mv: (b, 0, 0)),
            pl.BlockSpec((1, HID, FFN_D), lambda b, me, mv: (me[b], 0, 0)),
            pl.BlockSpec((1, 1, FFN_D), lambda b, me, mv: (me[b], 0, 0)),
            pl.BlockSpec((1, FFN_D, HID), lambda b, me, mv: (me[b], 0, 0)),
            pl.BlockSpec((1, 1, HID), lambda b, me, mv: (me[b], 0, 0)),
        ],
        out_specs=pl.BlockSpec((BLK, HID), lambda b, me, mv: (b, 0)),
    )
    return pl.pallas_call(
        _ffn_body,
        grid_spec=grid_spec,
        out_shape=jax.ShapeDtypeStruct((S_EXP, HID), jnp.float32),
    )(me, mv, xg, ggrp3, W1, b13, W2, b23)


def _combine_body(p_hbm, yg_hbm, ys_hbm, out_hbm,
                  idx0_v, idx1_v, r0, r1, r2, sem0, sem1):
    c = lax.axis_index("c")
    s = lax.axis_index("s")
    wid = s * NCC + c
    t0 = wid * TPW
    pltpu.sync_copy(p_hbm.at[pl.ds(t0, TPW)], idx0_v)
    pltpu.sync_copy(p_hbm.at[pl.ds(NTOK + t0, TPW)], idx1_v)
    for ci in range(TPW // CCH):
        a0 = pltpu.async_copy(
            yg_hbm.at[idx0_v.at[pl.ds(ci * CCH, CCH)]], r0, sem0)
        a1 = pltpu.async_copy(
            yg_hbm.at[idx1_v.at[pl.ds(ci * CCH, CCH)]], r1, sem1)
        pltpu.sync_copy(ys_hbm.at[pl.ds(t0 + ci * CCH, CCH)], r2)
        a0.wait()
        a1.wait()

        def addrow(rr, carry):
            for l in range(HID // 16):
                sl = pl.ds(l * 16, 16)
                r0[rr, sl] = r0[rr, sl] + r1[rr, sl] + r2[rr, sl]
            return carry
        lax.fori_loop(0, CCH, addrow, 0)
        pltpu.sync_copy(r0, out_hbm.at[pl.ds(t0 + ci * CCH, CCH)])


def _combine(p1d, yg, ys):
    mesh = plsc.VectorSubcoreMesh(core_axis_name="c", subcore_axis_name="s")
    return pl.kernel(
        _combine_body,
        out_type=jax.ShapeDtypeStruct((NTOK, HID), jnp.float32),
        mesh=mesh,
        scratch_types=[
            pltpu.VMEM((TPW,), jnp.int32),
            pltpu.VMEM((TPW,), jnp.int32),
            pltpu.VMEM((CCH, HID), jnp.float32),
            pltpu.VMEM((CCH, HID), jnp.float32),
            pltpu.VMEM((CCH, HID), jnp.float32),
            pltpu.SemaphoreType.DMA,
            pltpu.SemaphoreType.DMA,
        ],
        compiler_params=pltpu.CompilerParams(needs_layout_passes=False),
    )(p1d, yg, ys)


def kernel(hidden_states, Wg, W1, b1, W2, b2, Ws1, bs1, Ws2, bs2):
    orig = hidden_states.shape
    x = hidden_states.reshape(-1, orig[-1])
    b13 = b1[:, None, :]
    b23 = b2[:, None, :]

    pos, g, me, mv = _router(x, Wg)
    p1d = pos.reshape(NSLOT)
    g1d = g.reshape(NSLOT)
    xg, ggrp = _dispatch(p1d, g1d, x)
    ys = _shared(x, Ws1, bs1[None, :], Ws2, bs2[None, :])
    ggrp3 = ggrp.reshape(NBE, 1, BLK)
    yg = _ffn(me.reshape(NBE), mv.reshape(NBE), xg, ggrp3, W1, b13, W2, b23)
    out = _combine(p1d, yg, ys)
    return out.reshape(orig)
